# baseline (speedup 1.0000x reference)
import functools

import jax
import jax.numpy as jnp
from jax import lax
from jax.experimental import pallas as pl
from jax.experimental.pallas import tpu as pltpu
from jax.experimental.pallas import tpu_sc as plsc

_N_TABLES = 8
_VOCAB = 100000
_DIM = 64
_BATCH = 4096
_TOTAL = _N_TABLES * _BATCH

_info = plsc.get_sparse_core_info()
_NC, _NS, _L = _info.num_cores, _info.num_subcores, _info.num_lanes
_NW = _NC * _NS
_W_PER_TABLE = _NW // _N_TABLES
_D_PER_W = _DIM // _W_PER_TABLE

_SPLIT = 50048
_HI = _VOCAB - _SPLIT
_NCHUNK = _BATCH // _L


@functools.partial(
    pl.kernel,
    out_type=jax.ShapeDtypeStruct((_DIM, _TOTAL), jnp.float32),
    mesh=plsc.VectorSubcoreMesh(core_axis_name="c", subcore_axis_name="s"),
    scratch_types=[
        pltpu.VMEM((_BATCH,), jnp.int32),
        pltpu.VMEM((_SPLIT,), jnp.float32),
        pltpu.VMEM((_HI,), jnp.float32),
        pltpu.VMEM((_BATCH,), jnp.float32),
        pltpu.VMEM((_BATCH,), jnp.float32),
        pltpu.SemaphoreType.DMA,
        pltpu.SemaphoreType.DMA,
        pltpu.SemaphoreType.DMA,
        pltpu.SemaphoreType.DMA,
    ],
    compiler_params=pltpu.CompilerParams(
        use_tc_tiling_on_sc=True, needs_layout_passes=False
    ),
)
def _gather_kernel(
    ids_hbm, tables_hbm, out_hbm,
    ids_v, lo_v, hi_v, out0_v, out1_v,
    sem_lo, sem_hi, sem_o0, sem_o1,
):
    wid = lax.axis_index("s") * _NC + lax.axis_index("c")
    t = wid // _W_PER_TABLE
    d0 = (wid % _W_PER_TABLE) * _D_PER_W

    def _stage_lo(k):
        return pltpu.async_copy(
            tables_hbm.at[t, d0 + k, pl.ds(0, _SPLIT)], lo_v, sem_lo
        )

    def _stage_hi(k):
        return pltpu.async_copy(
            tables_hbm.at[t, d0 + k, pl.ds(_SPLIT, _HI)], hi_v, sem_hi
        )

    cp_lo = _stage_lo(0)
    cp_hi = _stage_hi(0)
    pltpu.sync_copy(ids_hbm.at[t], ids_v)

    def _pass_lo(out_v):
        def _body(i):
            idx16 = ids_v[pl.ds(i * _L, _L)]
            j = jnp.minimum(idx16, _SPLIT - 1)
            out_v[pl.ds(i * _L, _L)] = plsc.load_gather(lo_v, [j])

        plsc.parallel_loop(0, _NCHUNK, 1, unroll=16)(_body)

    def _pass_hi(out_v):
        def _body(i):
            sl = pl.ds(i * _L, _L)
            idx16 = ids_v[sl]
            j = jnp.maximum(idx16 - _SPLIT, 0)
            vals_hi = plsc.load_gather(hi_v, [j])
            out_v[sl] = jnp.where(idx16 < _SPLIT, out_v[sl], vals_hi)

        plsc.parallel_loop(0, _NCHUNK, 1, unroll=16)(_body)

    out_bufs = (out0_v, out1_v)
    out_sems = (sem_o0, sem_o1)
    out_copies = [None, None]

    for k in range(_D_PER_W):
        out_v = out_bufs[k % 2]
        if out_copies[k % 2] is not None:
            out_copies[k % 2].wait()
        cp_lo.wait()
        _pass_lo(out_v)
        if k + 1 < _D_PER_W:
            cp_lo = _stage_lo(k + 1)
        cp_hi.wait()
        _pass_hi(out_v)
        if k + 1 < _D_PER_W:
            cp_hi = _stage_hi(k + 1)
        out_copies[k % 2] = pltpu.async_copy(
            out_v, out_hbm.at[d0 + k, pl.ds(t * _BATCH, _BATCH)], out_sems[k % 2]
        )
    for c in out_copies:
        c.wait()


def kernel(ids, tables):
    tables_t = tables.transpose(0, 2, 1)
    out_t = _gather_kernel(ids, tables_t)
    return out_t.T


# speedup vs baseline: 1.0280x; 1.0280x over previous
import functools

import jax
import jax.numpy as jnp
from jax import lax
from jax.experimental import pallas as pl
from jax.experimental.pallas import tpu as pltpu
from jax.experimental.pallas import tpu_sc as plsc

_N_TABLES = 8
_VOCAB = 100000
_DIM = 64
_BATCH = 4096
_TOTAL = _N_TABLES * _BATCH

_info = plsc.get_sparse_core_info()
_NC, _NS, _L = _info.num_cores, _info.num_subcores, _info.num_lanes
_NW = _NC * _NS
_W_PER_TABLE = _NW // _N_TABLES
_D_PER_W = _DIM // _W_PER_TABLE

_SPLIT = 50048
_HI = _VOCAB - _SPLIT
_NCHUNK = _BATCH // _L


@functools.partial(
    pl.kernel,
    out_type=jax.ShapeDtypeStruct((_DIM, _TOTAL), jnp.float32),
    mesh=plsc.VectorSubcoreMesh(core_axis_name="c", subcore_axis_name="s"),
    scratch_types=[
        pltpu.VMEM((_BATCH,), jnp.int32),
        pltpu.VMEM((_SPLIT,), jnp.float32),
        pltpu.VMEM((_HI,), jnp.float32),
        pltpu.VMEM((_BATCH,), jnp.float32),
        pltpu.VMEM((_BATCH,), jnp.float32),
        pltpu.SemaphoreType.DMA,
        pltpu.SemaphoreType.DMA,
        pltpu.SemaphoreType.DMA,
        pltpu.SemaphoreType.DMA,
    ],
    compiler_params=pltpu.CompilerParams(
        use_tc_tiling_on_sc=True, needs_layout_passes=False
    ),
)
def _gather_kernel(
    ids_hbm, tables_hbm, out_hbm,
    ids_v, lo_v, hi_v, out0_v, out1_v,
    sem_lo, sem_hi, sem_o0, sem_o1,
):
    wid = lax.axis_index("s") * _NC + lax.axis_index("c")
    t = wid // _W_PER_TABLE
    d0 = (wid % _W_PER_TABLE) * _D_PER_W

    def _stage_lo(k):
        return pltpu.async_copy(
            tables_hbm.at[t, d0 + k, pl.ds(0, _SPLIT)], lo_v, sem_lo
        )

    def _stage_hi(k):
        return pltpu.async_copy(
            tables_hbm.at[t, d0 + k, pl.ds(_SPLIT, _HI)], hi_v, sem_hi
        )

    cp_lo = _stage_lo(0)
    cp_hi = _stage_hi(0)
    pltpu.sync_copy(ids_hbm.at[t], ids_v)

    def _pass_lo(out_v):
        def _body(i):
            idx16 = ids_v[pl.ds(i * _L, _L)]
            j = jnp.minimum(idx16, _SPLIT - 1)
            out_v[pl.ds(i * _L, _L)] = plsc.load_gather(lo_v, [j])

        plsc.parallel_loop(0, _NCHUNK, 1, unroll=4)(_body)

    def _pass_hi(out_v):
        def _body(i):
            sl = pl.ds(i * _L, _L)
            idx16 = ids_v[sl]
            j = jnp.maximum(idx16 - _SPLIT, 0)
            vals_hi = plsc.load_gather(hi_v, [j])
            out_v[sl] = jnp.where(idx16 < _SPLIT, out_v[sl], vals_hi)

        plsc.parallel_loop(0, _NCHUNK, 1, unroll=4)(_body)

    out_bufs = (out0_v, out1_v)
    out_sems = (sem_o0, sem_o1)
    out_copies = [None, None]

    for k in range(_D_PER_W):
        out_v = out_bufs[k % 2]
        if out_copies[k % 2] is not None:
            out_copies[k % 2].wait()
        cp_lo.wait()
        _pass_lo(out_v)
        if k + 1 < _D_PER_W:
            cp_lo = _stage_lo(k + 1)
        cp_hi.wait()
        _pass_hi(out_v)
        if k + 1 < _D_PER_W:
            cp_hi = _stage_hi(k + 1)
        out_copies[k % 2] = pltpu.async_copy(
            out_v, out_hbm.at[d0 + k, pl.ds(t * _BATCH, _BATCH)], out_sems[k % 2]
        )
    for c in out_copies:
        c.wait()


def kernel(ids, tables):
    tables_t = tables.transpose(0, 2, 1)
    out_t = _gather_kernel(ids, tables_t)
    return out_t.T


# speedup vs baseline: 1.0473x; 1.0187x over previous
import functools

import jax
import jax.numpy as jnp
from jax import lax
from jax.experimental import pallas as pl
from jax.experimental.pallas import tpu as pltpu
from jax.experimental.pallas import tpu_sc as plsc

_N_TABLES = 8
_VOCAB = 100000
_DIM = 64
_BATCH = 4096
_TOTAL = _N_TABLES * _BATCH

_info = plsc.get_sparse_core_info()
_NC, _NS, _L = _info.num_cores, _info.num_subcores, _info.num_lanes
_NW = _NC * _NS
_W_PER_TABLE = _NW // _N_TABLES
_D_PER_W = _DIM // _W_PER_TABLE

_SPLIT = 50048
_HI = _VOCAB - _SPLIT
_NCHUNK = _BATCH // _L


@functools.partial(
    pl.kernel,
    out_type=jax.ShapeDtypeStruct((_DIM, _TOTAL), jnp.float32),
    mesh=plsc.VectorSubcoreMesh(core_axis_name="c", subcore_axis_name="s"),
    scratch_types=[
        pltpu.VMEM((_BATCH,), jnp.int32),
        pltpu.VMEM((_VOCAB,), jnp.float32),
        pltpu.VMEM((_BATCH,), jnp.float32),
        pltpu.VMEM((_BATCH,), jnp.float32),
        pltpu.SemaphoreType.DMA,
        pltpu.SemaphoreType.DMA,
        pltpu.SemaphoreType.DMA,
        pltpu.SemaphoreType.DMA,
    ],
    compiler_params=pltpu.CompilerParams(
        use_tc_tiling_on_sc=True, needs_layout_passes=False
    ),
)
def _gather_kernel(
    ids_hbm, tables_hbm, out_hbm,
    ids_v, col_v, out0_v, out1_v,
    sem_lo, sem_hi, sem_o0, sem_o1,
):
    wid = lax.axis_index("s") * _NC + lax.axis_index("c")
    t = wid // _W_PER_TABLE
    d0 = (wid % _W_PER_TABLE) * _D_PER_W

    def _stage(k):
        return (pltpu.async_copy(tables_hbm.at[t, d0 + k], col_v, sem_lo),)

    cps = _stage(0)
    pltpu.sync_copy(ids_hbm.at[t], ids_v)

    def _gather(out_v):
        def _body(i):
            sl = pl.ds(i * _L, _L)
            out_v[sl] = plsc.load_gather(col_v, [ids_v[sl]])

        plsc.parallel_loop(0, _NCHUNK, 1, unroll=4)(_body)

    out_bufs = (out0_v, out1_v)
    out_sems = (sem_o0, sem_o1)
    out_copies = [None, None]

    for k in range(_D_PER_W):
        out_v = out_bufs[k % 2]
        if out_copies[k % 2] is not None:
            out_copies[k % 2].wait()
        for cp in cps:
            cp.wait()
        _gather(out_v)
        if k + 1 < _D_PER_W:
            cps = _stage(k + 1)
        out_copies[k % 2] = pltpu.async_copy(
            out_v, out_hbm.at[d0 + k, pl.ds(t * _BATCH, _BATCH)], out_sems[k % 2]
        )
    for c in out_copies:
        c.wait()


def kernel(ids, tables):
    tables_t = tables.transpose(0, 2, 1)
    out_t = _gather_kernel(ids, tables_t)
    return out_t.T
